# Initial kernel scaffold; baseline (speedup 1.0000x reference)
import jax
import jax.numpy as jnp
from jax.experimental import pallas as pl


def kernel(inputs, pos_embedding):
    raise NotImplementedError("write your pallas kernel here")


# speedup vs baseline: 5.0512x; 5.0512x over previous
import jax
import jax.numpy as jnp
from jax.experimental import pallas as pl


def _bcast_body(emb_ref, out_ref):
    out_ref[...] = jnp.broadcast_to(emb_ref[...][None], out_ref.shape)


def kernel(inputs, pos_embedding):
    B, seq_len = inputs.shape
    D = pos_embedding.shape[1]
    table = pos_embedding[:seq_len]
    BLK = 512
    nblk = seq_len // BLK
    return pl.pallas_call(
        _bcast_body,
        grid=(nblk,),
        in_specs=[pl.BlockSpec((BLK, D), lambda i: (i, 0))],
        out_specs=pl.BlockSpec((B, BLK, D), lambda i: (0, i, 0)),
        out_shape=jax.ShapeDtypeStruct((B, seq_len, D), pos_embedding.dtype),
    )(table)
